# Initial kernel scaffold; baseline (speedup 1.0000x reference)
import jax
import jax.numpy as jnp
from jax.experimental import pallas as pl


def kernel(x, params, edge_index, batch):
    raise NotImplementedError("write your pallas kernel here")


# speedup vs baseline: 5.2457x; 5.2457x over previous
import functools

import jax
import jax.numpy as jnp
from jax import lax
from jax.experimental import pallas as pl
from jax.experimental.pallas import tpu as pltpu
from jax.experimental.pallas import tpu_sc as plsc

_N = 10000
_G = 64
_E = 320000
_K = 128
_CPT = 80
_NT = 32
_EPAD = _NT * _CPT * _K
_NPAD = 10240
_RPT = _NPAD // 16
_BN_EPS = 1e-5


@functools.lru_cache(maxsize=None)
def _sc_aggregate(D):
    mesh = plsc.VectorSubcoreMesh(core_axis_name="c", subcore_axis_name="s")

    def body(h_hbm, src_hbm, dst_hbm, zeros_hbm, out_hbm,
             idx_s, idx_d, rows, agg_sh, sem):
        c = lax.axis_index("c")
        s = lax.axis_index("s")
        wid = c * 16 + s
        r0 = s * _RPT
        pltpu.sync_copy(zeros_hbm.at[pl.ds(r0, _RPT)], agg_sh.at[pl.ds(r0, _RPT)])
        pltpu.sync_copy(src_hbm.at[pl.ds(wid * _CPT, _CPT)], idx_s)
        pltpu.sync_copy(dst_hbm.at[pl.ds(wid * _CPT, _CPT)], idx_d)
        plsc.subcore_barrier()

        def chunk(j, carry):
            pltpu.async_copy(h_hbm.at[idx_s.at[j]], rows, sem).wait()
            pltpu.sync_copy(rows, agg_sh.at[idx_d.at[j]], add=True)
            return carry

        lax.fori_loop(0, _CPT, chunk, 0)
        plsc.subcore_barrier()
        pltpu.sync_copy(agg_sh.at[pl.ds(r0, _RPT)],
                        out_hbm.at[c].at[pl.ds(r0, _RPT)])

    return pl.kernel(
        body,
        out_type=jax.ShapeDtypeStruct((2, _NPAD, D), jnp.float32),
        mesh=mesh,
        scratch_types=[
            pltpu.VMEM((_CPT, _K), jnp.int32),
            pltpu.VMEM((_CPT, _K), jnp.int32),
            pltpu.VMEM((_K, D), jnp.float32),
            pltpu.VMEM_SHARED((_NPAD, D), jnp.float32),
            pltpu.SemaphoreType.DMA,
        ],
        compiler_params=pltpu.CompilerParams(use_tc_tiling_on_sc=False),
    )


def _tc_layer(h, a0, a1, Wa, ba, Wb, bb, gamma, beta):
    def body(h_ref, a0_ref, a1_ref, Wa_ref, ba_ref, Wb_ref, bb_ref,
             g_ref, be_ref, o_ref):
        t = h_ref[...] + a0_ref[...] + a1_ref[...]
        t = jnp.dot(t, Wa_ref[...], preferred_element_type=jnp.float32) + ba_ref[...]
        t = jnp.maximum(t, 0.0)
        t = jnp.dot(t, Wb_ref[...], preferred_element_type=jnp.float32) + bb_ref[...]
        t = jnp.maximum(t, 0.0)
        mu = jnp.mean(t, axis=0, keepdims=True)
        var = jnp.mean((t - mu) ** 2, axis=0, keepdims=True)
        o_ref[...] = (g_ref[...] * (t - mu) / jnp.sqrt(var + _BN_EPS)
                      + be_ref[...])

    return pl.pallas_call(
        body,
        out_shape=jax.ShapeDtypeStruct((_N, Wb.shape[1]), jnp.float32),
    )(h, a0, a1, Wa, ba.reshape(1, -1), Wb, bb.reshape(1, -1),
      gamma.reshape(1, -1), beta.reshape(1, -1))


def _pool_fc(h, batch2d, fcW, fcb):
    def body(h_ref, b_ref, W_ref, bias_ref, o_ref):
        seg = (lax.broadcasted_iota(jnp.int32, (_G, _N), 0)
               == b_ref[...]).astype(jnp.float32)
        pooled = jnp.dot(seg, h_ref[...], preferred_element_type=jnp.float32,
                         precision=lax.Precision.HIGHEST)
        o_ref[...] = jnp.maximum(
            jnp.dot(pooled, W_ref[...], preferred_element_type=jnp.float32)
            + bias_ref[...], 0.0)

    return pl.pallas_call(
        body,
        out_shape=jax.ShapeDtypeStruct((_G, fcW.shape[1]), jnp.float32),
    )(h, batch2d, fcW, fcb.reshape(1, -1))


def kernel(x, params, edge_index, batch):
    src = edge_index[0]
    dst = edge_index[1]
    pad = _EPAD - _E
    srcp = jnp.concatenate([src, jnp.zeros((pad,), jnp.int32)]).reshape(-1, _K)
    dstp = jnp.concatenate([dst, jnp.full((pad,), _N, jnp.int32)]).reshape(-1, _K)
    batch2d = batch.reshape(1, _N)

    h = x
    for i in range(1, 6):
        D = h.shape[1]
        zeros = jnp.zeros((_NPAD, D), jnp.float32)
        agg = _sc_aggregate(D)(h, srcp, dstp, zeros)
        h = _tc_layer(h, agg[0, :_N], agg[1, :_N],
                      params['conv%d_Wa' % i], params['conv%d_ba' % i],
                      params['conv%d_Wb' % i], params['conv%d_bb' % i],
                      params['bn%d_gamma' % i], params['bn%d_beta' % i])
    return _pool_fc(h, batch2d, params['fc_W'], params['fc_b'])
